# Initial kernel scaffold; baseline (speedup 1.0000x reference)
import jax
import jax.numpy as jnp
from jax.experimental import pallas as pl


def kernel(x, timestamp, tok_table, time_table, gamma, beta):
    raise NotImplementedError("write your pallas kernel here")


# speedup vs baseline: 1.7747x; 1.7747x over previous
import functools

import numpy as np
import jax
import jax.numpy as jnp
from jax import lax
from jax.experimental import pallas as pl
from jax.experimental.pallas import tpu as pltpu
from jax.experimental.pallas import tpu_sc as plsc

_B = 4096
_L = 200
_D = 64
_EPS = 1e-5
_NC, _NS = 2, 16
_NW = _NC * _NS
_ROWS_PER_W = _B // _NW
_LANES = 16
_NV = _D // _LANES


def _make_pe_np(max_len, d):
    position = np.arange(max_len, dtype=np.float32)[:, None]
    div_term = np.exp(np.arange(0, d, 2, dtype=np.float32) * -(np.log(10000.0) / d))
    pe = np.zeros((max_len, d), dtype=np.float32)
    pe[:, 0::2] = np.sin(position * div_term)
    pe[:, 1::2] = np.cos(position * div_term)
    return pe


def _sc_body(x_hbm, t_hbm, tok_hbm, time_hbm, pe_hbm, gam_hbm, bet_hbm,
             out_hbm, xi_v, ti_v, tokr_v, timr_v, out_v, pe_v, gam_v, bet_v,
             sem):
    wid = lax.axis_index("s") * _NC + lax.axis_index("c")
    pltpu.sync_copy(pe_hbm, pe_v)
    pltpu.sync_copy(gam_hbm, gam_v)
    pltpu.sync_copy(bet_hbm, bet_v)
    base_row = wid * _ROWS_PER_W
    half = 1.5

    def per_row(r, carry):
        row = base_row + r
        pltpu.sync_copy(x_hbm.at[row], xi_v)
        pltpu.sync_copy(t_hbm.at[row], ti_v)
        cps = [
            pltpu.async_copy(tok_hbm.at[xi_v.at[0]], tokr_v.at[pl.ds(0, 100)], sem),
            pltpu.async_copy(tok_hbm.at[xi_v.at[1]], tokr_v.at[pl.ds(100, 100)], sem),
            pltpu.async_copy(time_hbm.at[ti_v.at[0]], timr_v.at[pl.ds(0, 100)], sem),
            pltpu.async_copy(time_hbm.at[ti_v.at[1]], timr_v.at[pl.ds(100, 100)], sem),
        ]
        for cp in cps:
            cp.wait()

        iota = lax.iota(jnp.int32, _LANES)

        def per_pos(i, c):
            vs = [tokr_v[i, pl.ds(k * _LANES, _LANES)]
                  + timr_v[i, pl.ds(k * _LANES, _LANES)]
                  + pe_v[i, pl.ds(k * _LANES, _LANES)]
                  for k in range(_NV)]
            s1 = (vs[0] + vs[1]) + (vs[2] + vs[3])
            s2 = (vs[0] * vs[0] + vs[1] * vs[1]) + (vs[2] * vs[2] + vs[3] * vs[3])
            for step in (1, 2, 4, 8):
                perm = iota ^ step
                s1 = s1 + jnp.take_along_axis(s1, perm, axis=0,
                                              mode="promise_in_bounds")
                s2 = s2 + jnp.take_along_axis(s2, perm, axis=0,
                                              mode="promise_in_bounds")
            mu = s1 * (1.0 / _D)
            var = s2 * (1.0 / _D) - mu * mu + _EPS
            yi = (jnp.int32(0x5F3759DF)
                  - (lax.bitcast_convert_type(var, jnp.int32) >> 1))
            y = lax.bitcast_convert_type(yi, jnp.float32)
            hx = var * 0.5
            y = y * (half - hx * y * y)
            y = y * (half - hx * y * y)
            y = y * (half - hx * y * y)
            for k in range(_NV):
                g = gam_v[pl.ds(k * _LANES, _LANES)]
                b = bet_v[pl.ds(k * _LANES, _LANES)]
                out_v[i, pl.ds(k * _LANES, _LANES)] = (vs[k] - mu) * y * g + b
            return c

        lax.fori_loop(0, _L, per_pos, 0)
        pltpu.sync_copy(out_v, out_hbm.at[row])
        return carry

    lax.fori_loop(0, _ROWS_PER_W, per_row, 0)


@jax.jit
def kernel(x, timestamp, tok_table, time_table, gamma, beta):
    pe = jnp.asarray(_make_pe_np(_L, _D))
    x2 = x.reshape(_B, 2, _L // 2)
    t2 = timestamp.reshape(_B, 2, _L // 2)
    mesh = plsc.VectorSubcoreMesh(core_axis_name="c", subcore_axis_name="s",
                                  num_cores=_NC, num_subcores=_NS)
    run = pl.kernel(
        _sc_body,
        out_type=jax.ShapeDtypeStruct((_B, _L, _D), jnp.float32),
        mesh=mesh,
        compiler_params=pltpu.CompilerParams(use_tc_tiling_on_sc=False),
        scratch_types=[
            pltpu.VMEM((2, _L // 2), jnp.int32),
            pltpu.VMEM((2, _L // 2), jnp.int32),
            pltpu.VMEM((_L, _D), jnp.float32),
            pltpu.VMEM((_L, _D), jnp.float32),
            pltpu.VMEM((_L, _D), jnp.float32),
            pltpu.VMEM((_L, _D), jnp.float32),
            pltpu.VMEM((_D,), jnp.float32),
            pltpu.VMEM((_D,), jnp.float32),
            pltpu.SemaphoreType.DMA,
        ],
    )
    return run(x2, t2, tok_table, time_table, pe, gamma, beta)
